# Initial kernel scaffold; baseline (speedup 1.0000x reference)
import jax
import jax.numpy as jnp
from jax import lax
from jax.experimental import pallas as pl
from jax.experimental.pallas import tpu as pltpu
from jax.experimental.pallas import tpu_sc as plsc

N = 10000
E = 320000
D = 128
BN_EPS = 1e-5

NC = 2
NS = 16

E_CORE = E // NC
E_SUB = E_CORE // NS
CHUNK = 128
N_FULL = E_SUB // CHUNK
TAIL = E_SUB - N_FULL * CHUNK
N_SUB = N // NS


def _sc_agg_body(x_hbm, row_hbm, col_hbm, z_hbm, out_hbm,
                 idx_r, idx_c, buf, idx_r2, idx_c2, buf2, agg_sh):
    c = lax.axis_index("c")
    s = lax.axis_index("s")

    pltpu.sync_copy(z_hbm, agg_sh.at[pl.ds(s * N_SUB, N_SUB)])
    plsc.subcore_barrier()

    base = c * E_CORE + s * E_SUB

    @pl.loop(0, N_FULL)
    def _(j):
        e0 = base + j * CHUNK
        pltpu.sync_copy(row_hbm.at[pl.ds(e0, CHUNK)], idx_r)
        pltpu.sync_copy(x_hbm.at[idx_r], buf)
        pltpu.sync_copy(col_hbm.at[pl.ds(e0, CHUNK)], idx_c)
        pltpu.sync_copy(buf, agg_sh.at[idx_c], add=True)

    if TAIL:
        e0 = base + N_FULL * CHUNK
        pltpu.sync_copy(row_hbm.at[pl.ds(e0, TAIL)], idx_r2)
        pltpu.sync_copy(x_hbm.at[idx_r2], buf2)
        pltpu.sync_copy(col_hbm.at[pl.ds(e0, TAIL)], idx_c2)
        pltpu.sync_copy(buf2, agg_sh.at[idx_c2], add=True)

    plsc.subcore_barrier()
    pltpu.sync_copy(agg_sh.at[pl.ds(s * N_SUB, N_SUB)],
                    out_hbm.at[c, pl.ds(s * N_SUB, N_SUB)])


def _sc_aggregate(x, row, col, zeros_block):
    mesh = plsc.VectorSubcoreMesh(core_axis_name="c", subcore_axis_name="s",
                                  num_cores=NC, num_subcores=NS)
    kern = pl.kernel(
        _sc_agg_body,
        out_type=jax.ShapeDtypeStruct((NC, N, D), jnp.float32),
        mesh=mesh,
        scratch_types=[
            pltpu.VMEM((CHUNK,), jnp.int32),
            pltpu.VMEM((CHUNK,), jnp.int32),
            pltpu.VMEM((CHUNK, D), jnp.float32),
            pltpu.VMEM((TAIL,), jnp.int32),
            pltpu.VMEM((TAIL,), jnp.int32),
            pltpu.VMEM((TAIL, D), jnp.float32),
            pltpu.VMEM_SHARED((N, D), jnp.float32),
        ],
    )
    return kern(x, row, col, zeros_block)


def _mlp_body(eps_ref, x_ref, a0_ref, a1_ref, w1_ref, b1_ref, g1_ref,
              be1_ref, w2_ref, b2_ref, g2_ref, be2_ref, o_ref):
    h = x_ref[...] * (1.0 + eps_ref[0]) + a0_ref[...] + a1_ref[...]
    h = jnp.dot(h, w1_ref[...], preferred_element_type=jnp.float32)
    h = h + b1_ref[...]
    m = jnp.mean(h, axis=0, keepdims=True)
    hc = h - m
    v = jnp.mean(hc * hc, axis=0, keepdims=True)
    h = hc * lax.rsqrt(v + BN_EPS) * g1_ref[...] + be1_ref[...]
    h = jnp.maximum(h, 0.0)
    h = jnp.dot(h, w2_ref[...], preferred_element_type=jnp.float32)
    h = h + b2_ref[...]
    m2 = jnp.mean(h, axis=0, keepdims=True)
    hc2 = h - m2
    v2 = jnp.mean(hc2 * hc2, axis=0, keepdims=True)
    o_ref[...] = hc2 * lax.rsqrt(v2 + BN_EPS) * g2_ref[...] + be2_ref[...]


def _mlp(eps, x, a0, a1, W1, b1, g1, be1, W2, b2, g2, be2):
    smem_spec = pl.BlockSpec(memory_space=pltpu.SMEM)
    vmem_spec = pl.BlockSpec(memory_space=pltpu.VMEM)
    return pl.pallas_call(
        _mlp_body,
        out_shape=jax.ShapeDtypeStruct((N, D), jnp.float32),
        in_specs=[smem_spec] + [vmem_spec] * 11,
        out_specs=vmem_spec,
    )(eps, x, a0, a1, W1, b1, g1, be1, W2, b2, g2, be2)


@jax.jit
def kernel(x, edge_index, W1, b1, g1, be1, W2, b2, g2, be2, eps):
    row = edge_index[0]
    col = edge_index[1]
    zeros_block = jnp.zeros((N_SUB, D), jnp.float32)
    parts = _sc_aggregate(x, row, col, zeros_block)
    return _mlp(eps, x, parts[0], parts[1],
                W1, b1.reshape(1, D), g1.reshape(1, D), be1.reshape(1, D),
                W2, b2.reshape(1, D), g2.reshape(1, D), be2.reshape(1, D))


# speedup vs baseline: 5.9129x; 5.9129x over previous
import jax
import jax.numpy as jnp
from jax import lax
from jax.experimental import pallas as pl
from jax.experimental.pallas import tpu as pltpu
from jax.experimental.pallas import tpu_sc as plsc

N = 10000
E = 320000
D = 128
BN_EPS = 1e-5

NC = 2
NS = 16

E_CORE = E // NC
E_SUB = E_CORE // NS
CHUNK = 128
N_FULL = E_SUB // CHUNK
TAIL = E_SUB - N_FULL * CHUNK
N_SUB = 624
N_REM = N - NS * N_SUB


def _sc_agg_body(x_hbm, row_hbm, col_hbm, z_hbm, out_hbm,
                 idx_r, idx_c, buf, idx_r2, idx_c2, buf2, agg_sh):
    c = lax.axis_index("c")
    s = lax.axis_index("s")

    pltpu.sync_copy(z_hbm, agg_sh.at[pl.ds(s * N_SUB, N_SUB)])

    @pl.when(s == 0)
    def _():
        pltpu.sync_copy(z_hbm.at[pl.ds(0, N_REM)],
                        agg_sh.at[pl.ds(NS * N_SUB, N_REM)])

    plsc.subcore_barrier()

    base = c * E_CORE + s * E_SUB

    @pl.loop(0, N_FULL)
    def _(j):
        e0 = base + j * CHUNK
        pltpu.sync_copy(row_hbm.at[pl.ds(e0, CHUNK)], idx_r)
        pltpu.sync_copy(x_hbm.at[idx_r], buf)
        pltpu.sync_copy(col_hbm.at[pl.ds(e0, CHUNK)], idx_c)
        pltpu.sync_copy(buf, agg_sh.at[idx_c], add=True)

    if TAIL:
        e0 = base + N_FULL * CHUNK
        pltpu.sync_copy(row_hbm.at[pl.ds(e0, TAIL)], idx_r2)
        pltpu.sync_copy(x_hbm.at[idx_r2], buf2)
        pltpu.sync_copy(col_hbm.at[pl.ds(e0, TAIL)], idx_c2)
        pltpu.sync_copy(buf2, agg_sh.at[idx_c2], add=True)

    plsc.subcore_barrier()
    pltpu.sync_copy(agg_sh.at[pl.ds(s * N_SUB, N_SUB)],
                    out_hbm.at[c, pl.ds(s * N_SUB, N_SUB)])

    @pl.when(s == 0)
    def _():
        pltpu.sync_copy(agg_sh.at[pl.ds(NS * N_SUB, N_REM)],
                        out_hbm.at[c, pl.ds(NS * N_SUB, N_REM)])


def _sc_aggregate(x, row, col, zeros_block):
    mesh = plsc.VectorSubcoreMesh(core_axis_name="c", subcore_axis_name="s",
                                  num_cores=NC, num_subcores=NS)
    kern = pl.kernel(
        _sc_agg_body,
        out_type=jax.ShapeDtypeStruct((NC, N, D), jnp.float32),
        mesh=mesh,
        scratch_types=[
            pltpu.VMEM((CHUNK,), jnp.int32),
            pltpu.VMEM((CHUNK,), jnp.int32),
            pltpu.VMEM((CHUNK, D), jnp.float32),
            pltpu.VMEM((TAIL,), jnp.int32),
            pltpu.VMEM((TAIL,), jnp.int32),
            pltpu.VMEM((TAIL, D), jnp.float32),
            pltpu.VMEM_SHARED((N, D), jnp.float32),
        ],
    )
    return kern(x, row, col, zeros_block)


def _mlp_body(eps_ref, x_ref, a0_ref, a1_ref, w1_ref, b1_ref, g1_ref,
              be1_ref, w2_ref, b2_ref, g2_ref, be2_ref, o_ref):
    h = x_ref[...] * (1.0 + eps_ref[0]) + a0_ref[...] + a1_ref[...]
    h = jnp.dot(h, w1_ref[...], preferred_element_type=jnp.float32)
    h = h + b1_ref[...]
    m = jnp.mean(h, axis=0, keepdims=True)
    hc = h - m
    v = jnp.mean(hc * hc, axis=0, keepdims=True)
    h = hc * lax.rsqrt(v + BN_EPS) * g1_ref[...] + be1_ref[...]
    h = jnp.maximum(h, 0.0)
    h = jnp.dot(h, w2_ref[...], preferred_element_type=jnp.float32)
    h = h + b2_ref[...]
    m2 = jnp.mean(h, axis=0, keepdims=True)
    hc2 = h - m2
    v2 = jnp.mean(hc2 * hc2, axis=0, keepdims=True)
    o_ref[...] = hc2 * lax.rsqrt(v2 + BN_EPS) * g2_ref[...] + be2_ref[...]


def _mlp(eps, x, a0, a1, W1, b1, g1, be1, W2, b2, g2, be2):
    smem_spec = pl.BlockSpec(memory_space=pltpu.SMEM)
    vmem_spec = pl.BlockSpec(memory_space=pltpu.VMEM)
    return pl.pallas_call(
        _mlp_body,
        out_shape=jax.ShapeDtypeStruct((N, D), jnp.float32),
        in_specs=[smem_spec] + [vmem_spec] * 11,
        out_specs=vmem_spec,
    )(eps, x, a0, a1, W1, b1, g1, be1, W2, b2, g2, be2)


@jax.jit
def kernel(x, edge_index, W1, b1, g1, be1, W2, b2, g2, be2, eps):
    row = edge_index[0]
    col = edge_index[1]
    zeros_block = jnp.zeros((N_SUB, D), jnp.float32)
    parts = _sc_aggregate(x, row, col, zeros_block)
    return _mlp(eps, x, parts[0], parts[1],
                W1, b1.reshape(1, D), g1.reshape(1, D), be1.reshape(1, D),
                W2, b2.reshape(1, D), g2.reshape(1, D), be2.reshape(1, D))
